# baseline (speedup 1.0000x reference)
import functools

import jax
import jax.numpy as jnp
from jax import lax
from jax.experimental import pallas as pl
from jax.experimental.pallas import tpu as pltpu
from jax.experimental.pallas import tpu_sc as plsc

SLOPE = 11.0 / 48.0
NS = 16
CH = 80
DEGW = 16


def _sc_main(edge_feats, dst_idx, table, src_idx, zrows, ones_feat, NP):
    E, D = edge_feats.shape
    HN = NP // 2
    RPH = HN // NS
    EPW = E // NS
    NCH = EPW // CH
    mesh = plsc.VectorSubcoreMesh(core_axis_name="c", subcore_axis_name="s",
                                  num_cores=1)

    @functools.partial(
        pl.kernel,
        out_type=jax.ShapeDtypeStruct((3 * NP, D), jnp.float32),
        mesh=mesh,
        scratch_types=[
            pltpu.VMEM((RPH, D), jnp.float32),
            pltpu.VMEM((CH, D), jnp.float32),
            pltpu.VMEM((CH, D), jnp.float32),
            pltpu.VMEM((CH,), jnp.int32),
            pltpu.VMEM((CH,), jnp.int32),
            pltpu.VMEM((CH,), jnp.int32),
            pltpu.VMEM_SHARED((HN + 8, D), jnp.float32),
        ],
    )
    def k(ef_hbm, dst_hbm, tab_hbm, src_hbm, zr_hbm, on_hbm, out,
          zbuf, rows_v, ones_v, src_v, dst_v, loc_v, acc):
        s = lax.axis_index("s")
        r0 = s * RPH
        base = s * EPW
        pltpu.sync_copy(zr_hbm, zbuf)
        pltpu.sync_copy(on_hbm, ones_v)

        def remap(lo):
            for j in range(CH // 16):
                v = dst_v[pl.ds(j * 16, 16)]
                loc = v - lo
                ok = (loc >= 0) & (loc < HN)
                loc_v[pl.ds(j * 16, 16)] = jnp.where(
                    ok, loc, jnp.full((16,), HN, jnp.int32))

        for half in range(2):
            lo = half * HN
            pltpu.sync_copy(zbuf, acc.at[pl.ds(r0, RPH)])
            plsc.subcore_barrier()

            def p1_body(i, carry):
                off = base + i * CH
                pltpu.sync_copy(dst_hbm.at[pl.ds(off, CH)], dst_v)
                remap(lo)
                pltpu.sync_copy(ef_hbm.at[pl.ds(off, CH)], rows_v)
                pltpu.sync_copy(rows_v, acc.at[loc_v], add=True)
                return carry

            lax.fori_loop(0, NCH, p1_body, 0)
            plsc.subcore_barrier()
            pltpu.sync_copy(acc.at[pl.ds(r0, RPH)],
                            out.at[pl.ds(lo + r0, RPH)])
            pltpu.sync_copy(zbuf, acc.at[pl.ds(r0, RPH)])
            plsc.subcore_barrier()

            def p2_body(i, carry):
                off = base + i * CH
                pltpu.sync_copy(src_hbm.at[pl.ds(off, CH)], src_v)
                pltpu.sync_copy(dst_hbm.at[pl.ds(off, CH)], dst_v)
                remap(lo)
                pltpu.sync_copy(tab_hbm.at[src_v], rows_v)
                pltpu.sync_copy(rows_v, acc.at[loc_v], add=True)
                return carry

            lax.fori_loop(0, NCH, p2_body, 0)
            plsc.subcore_barrier()
            pltpu.sync_copy(acc.at[pl.ds(r0, RPH)],
                            out.at[pl.ds(NP + lo + r0, RPH)])
            pltpu.sync_copy(zbuf, acc.at[pl.ds(r0, RPH)])
            plsc.subcore_barrier()

            def p3_body(i, carry):
                off = base + i * CH
                pltpu.sync_copy(dst_hbm.at[pl.ds(off, CH)], dst_v)
                remap(lo)
                pltpu.sync_copy(ones_v, acc.at[loc_v], add=True)
                return carry

            lax.fori_loop(0, NCH, p3_body, 0)
            plsc.subcore_barrier()
            pltpu.sync_copy(acc.at[pl.ds(r0, RPH)],
                            out.at[pl.ds(2 * NP + lo + r0, RPH)])
            plsc.subcore_barrier()

    return k(edge_feats, dst_idx, table, src_idx, zrows, ones_feat)


def _tc_layer(gx, ge, dg, W1t, W1b, b1, W2, b2, N, R=400):
    D = ge.shape[1]
    G = N // R

    def body(gx_r, ge_r, dg_r, w1t_r, w1b_r, b1_r, w2_r, b2_r, out_r):
        deg = dg_r[:, 0:1]
        agg = (jnp.dot(gx_r[...], w1t_r[...], preferred_element_type=jnp.float32)
               + jnp.dot(ge_r[...], w1b_r[...], preferred_element_type=jnp.float32)
               + deg * b1_r[...])
        mean = agg / jnp.maximum(deg, 1.0)
        out = (mean + jnp.dot(mean, w2_r[...], preferred_element_type=jnp.float32)
               + b2_r[...])
        out_r[...] = jnp.where(out >= 0, out, SLOPE * out)

    x_spec = pl.BlockSpec((R, D), lambda i: (i, 0))
    w_spec = pl.BlockSpec((D, D), lambda i: (0, 0))
    b_spec = pl.BlockSpec((1, D), lambda i: (0, 0))
    return pl.pallas_call(
        body,
        grid=(G,),
        in_specs=[x_spec, x_spec, x_spec, w_spec, w_spec, b_spec,
                  w_spec, b_spec],
        out_specs=pl.BlockSpec((R, D), lambda i: (i, 0)),
        out_shape=jax.ShapeDtypeStruct((N, D), jnp.float32),
    )(gx, ge, dg, W1t, W1b, b1.reshape(1, D), W2, b2.reshape(1, D))


def _tc_final(h0, h1, h2, Wf0, Wf1, Wf2, bf, R=400):
    N, D = h0.shape
    G = N // R

    def body(h0_r, h1_r, h2_r, wf0_r, wf1_r, wf2_r, bf_r, out_r):
        out_r[...] = (jnp.dot(h0_r[...], wf0_r[...], preferred_element_type=jnp.float32)
                      + jnp.dot(h1_r[...], wf1_r[...], preferred_element_type=jnp.float32)
                      + jnp.dot(h2_r[...], wf2_r[...], preferred_element_type=jnp.float32)
                      + bf_r[...])

    x_spec = pl.BlockSpec((R, D), lambda i: (i, 0))
    w_spec = pl.BlockSpec((D, D), lambda i: (0, 0))
    b_spec = pl.BlockSpec((1, D), lambda i: (0, 0))
    return pl.pallas_call(
        body,
        grid=(G,),
        in_specs=[x_spec, x_spec, x_spec, w_spec, w_spec, w_spec, b_spec],
        out_specs=pl.BlockSpec((R, D), lambda i: (i, 0)),
        out_shape=jax.ShapeDtypeStruct((N, D), jnp.float32),
    )(h0, h1, h2, Wf0, Wf1, Wf2, bf.reshape(1, D))


def kernel(node_feats, edge_feats, edge_index, W1_0, b1_0, W2_0, b2_0,
           W1_1, b1_1, W2_1, b2_1, Wf, bf):
    N, D = node_feats.shape
    src = edge_index[0]
    dst = edge_index[1]
    NP = ((N + 255) // 256) * 256
    zrows = jnp.zeros((NP // 2 // NS, D), jnp.float32)
    ones_feat = jnp.ones((CH, D), jnp.float32)

    w1t = jnp.stack([W1_0[:D], W1_1[:D]])
    w1b = jnp.stack([W1_0[D:], W1_1[D:]])
    b1s = jnp.stack([b1_0, b1_1])
    w2s = jnp.stack([W2_0, W2_1])
    b2s = jnp.stack([b2_0, b2_1])

    def step(h, ws):
        w1t_i, w1b_i, b1_i, w2_i, b2_i = ws
        gegx = _sc_main(edge_feats, dst, h, src, zrows, ones_feat, NP)
        ge = gegx[:NP]
        gx = gegx[NP:2 * NP]
        dg = gegx[2 * NP:]
        h_next = _tc_layer(gx, ge, dg, w1t_i, w1b_i, b1_i, w2_i, b2_i, N)
        return h_next, h_next

    _, hs = lax.scan(step, node_feats, (w1t, w1b, b1s, w2s, b2s))
    h1, h2 = hs[0], hs[1]
    return _tc_final(node_feats, h1, h2, Wf[:D], Wf[D:2 * D], Wf[2 * D:], bf)


# speedup vs baseline: 1.8501x; 1.8501x over previous
import functools

import jax
import jax.numpy as jnp
from jax import lax
from jax.experimental import pallas as pl
from jax.experimental.pallas import tpu as pltpu
from jax.experimental.pallas import tpu_sc as plsc

SLOPE = 11.0 / 48.0
NS = 16
CH = 80
DEGW = 16


def _sc_main(edge_feats, dst3d, table, src3d, zrows, ones_feat, flag, NP):
    E, D = edge_feats.shape
    HN = NP // 2
    RPH = HN // NS
    NCH = dst3d.shape[0] * dst3d.shape[1] // NS
    SCH = dst3d.shape[1]
    NCHL = (E - (NS - 1) * NCH * CH) // CH
    mesh = plsc.VectorSubcoreMesh(core_axis_name="c", subcore_axis_name="s",
                                  num_cores=1)

    @functools.partial(
        pl.kernel,
        out_type=jax.ShapeDtypeStruct((3 * NP, D), jnp.float32),
        mesh=mesh,
        scratch_types=[
            pltpu.VMEM((16, D), jnp.float32),
            pltpu.VMEM((CH, D), jnp.float32),
            pltpu.VMEM((SCH, CH), jnp.int32),
            pltpu.VMEM((SCH, CH), jnp.int32),
            pltpu.VMEM((16,), jnp.int32),
            pltpu.VMEM_SHARED((HN + 8, D), jnp.float32),
        ],
    )
    def k(ef_hbm, dst_hbm, tab_hbm, src_hbm, zr_hbm, on_hbm, fl_hbm, out,
          zbuf, pool, dst_v, src_v, fl_v, acc):
        s = lax.axis_index("s")
        r0 = s * RPH
        nch1 = lax.select(s < NS - 1, jnp.int32(NCH), jnp.int32(NCHL))
        pltpu.sync_copy(zr_hbm, zbuf)
        pltpu.sync_copy(fl_hbm, fl_v)
        do_e = fl_v[...][0] > 0

        def zero_acc():
            for q in range(RPH // 16):
                pltpu.sync_copy(zbuf, acc.at[pl.ds(r0 + q * 16, 16)])

        def copyout(sect, lo):
            pltpu.sync_copy(acc.at[pl.ds(r0, RPH)],
                            out.at[pl.ds(sect * NP + lo + r0, RPH)])

        def stage_dst(st, lo):
            pltpu.sync_copy(dst_hbm.at[s * 2 + st], dst_v)

            def rbody(i, carry):
                for j in range(CH // 16):
                    v = dst_v[i, pl.ds(j * 16, 16)]
                    loc = v - lo
                    ok = (loc >= 0) & (loc < HN)
                    dst_v[i, pl.ds(j * 16, 16)] = jnp.where(
                        ok, loc, jnp.full((16,), HN, jnp.int32))
                return carry

            lax.fori_loop(0, SCH, rbody, 0)

        def pipeline(load_fn, n):
            def body(i, carry):
                load_fn(i)
                pltpu.sync_copy(pool, acc.at[dst_v.at[i]], add=True)
                return carry

            lax.fori_loop(0, n, body, 0)

        for half in range(2):
            lo = half * HN

            @pl.when(do_e)
            def _():
                zero_acc()
                plsc.subcore_barrier()
                for st in range(2):
                    cb = st * SCH
                    stage_dst(st, lo)
                    nst = jnp.clip(nch1 - cb, 0, SCH)

                    def load_edge(c):
                        off = (s * NCH + cb) * CH + c * CH
                        pltpu.sync_copy(ef_hbm.at[pl.ds(off, CH)], pool)

                    pipeline(load_edge, nst)
                plsc.subcore_barrier()
                copyout(0, lo)
                plsc.subcore_barrier()

            zero_acc()
            plsc.subcore_barrier()
            for st in range(2):
                stage_dst(st, lo)
                pltpu.sync_copy(src_hbm.at[s * 2 + st], src_v)

                def load_gather(c):
                    pltpu.sync_copy(tab_hbm.at[src_v.at[c]], pool)

                pipeline(load_gather, jnp.int32(SCH))
            plsc.subcore_barrier()
            copyout(1, lo)
            plsc.subcore_barrier()

            @pl.when(do_e)
            def _():
                zero_acc()
                pltpu.sync_copy(on_hbm, pool)
                plsc.subcore_barrier()
                for st in range(2):
                    stage_dst(st, lo)

                    def p3(i, carry):
                        pltpu.sync_copy(pool, acc.at[dst_v.at[i]], add=True)
                        return carry

                    lax.fori_loop(0, SCH, p3, 0)
                plsc.subcore_barrier()
                copyout(2, lo)
                plsc.subcore_barrier()

    return k(edge_feats, dst3d, table, src3d, zrows, ones_feat, flag)


def _tc_layer(gx, ge, dg, W1t, W1b, b1, W2, b2, N, R=400):
    D = ge.shape[1]
    G = N // R

    def body(gx_r, ge_r, dg_r, w1t_r, w1b_r, b1_r, w2_r, b2_r, out_r):
        deg = dg_r[:, 0:1]
        agg = (jnp.dot(gx_r[...], w1t_r[...], preferred_element_type=jnp.float32)
               + jnp.dot(ge_r[...], w1b_r[...], preferred_element_type=jnp.float32)
               + deg * b1_r[...])
        mean = agg / jnp.maximum(deg, 1.0)
        out = (mean + jnp.dot(mean, w2_r[...], preferred_element_type=jnp.float32)
               + b2_r[...])
        out_r[...] = jnp.where(out >= 0, out, SLOPE * out)

    x_spec = pl.BlockSpec((R, D), lambda i: (i, 0))
    w_spec = pl.BlockSpec((D, D), lambda i: (0, 0))
    b_spec = pl.BlockSpec((1, D), lambda i: (0, 0))
    return pl.pallas_call(
        body,
        grid=(G,),
        in_specs=[x_spec, x_spec, x_spec, w_spec, w_spec, b_spec,
                  w_spec, b_spec],
        out_specs=pl.BlockSpec((R, D), lambda i: (i, 0)),
        out_shape=jax.ShapeDtypeStruct((N, D), jnp.float32),
    )(gx, ge, dg, W1t, W1b, b1.reshape(1, D), W2, b2.reshape(1, D))


def _tc_final(h0, h1, h2, Wf0, Wf1, Wf2, bf, R=400):
    N, D = h0.shape
    G = N // R

    def body(h0_r, h1_r, h2_r, wf0_r, wf1_r, wf2_r, bf_r, out_r):
        out_r[...] = (jnp.dot(h0_r[...], wf0_r[...], preferred_element_type=jnp.float32)
                      + jnp.dot(h1_r[...], wf1_r[...], preferred_element_type=jnp.float32)
                      + jnp.dot(h2_r[...], wf2_r[...], preferred_element_type=jnp.float32)
                      + bf_r[...])

    x_spec = pl.BlockSpec((R, D), lambda i: (i, 0))
    w_spec = pl.BlockSpec((D, D), lambda i: (0, 0))
    b_spec = pl.BlockSpec((1, D), lambda i: (0, 0))
    return pl.pallas_call(
        body,
        grid=(G,),
        in_specs=[x_spec, x_spec, x_spec, w_spec, w_spec, w_spec, b_spec],
        out_specs=pl.BlockSpec((R, D), lambda i: (i, 0)),
        out_shape=jax.ShapeDtypeStruct((N, D), jnp.float32),
    )(h0, h1, h2, Wf0, Wf1, Wf2, bf.reshape(1, D))


def kernel(node_feats, edge_feats, edge_index, W1_0, b1_0, W2_0, b2_0,
           W1_1, b1_1, W2_1, b2_1, Wf, bf):
    N, D = node_feats.shape
    E = edge_feats.shape[0]
    src = edge_index[0]
    dst = edge_index[1]
    NP = ((N + 255) // 256) * 256
    EPW = -(-E // (NS * CH * 2)) * (CH * 2)
    pad = NS * EPW - E
    srcp = jnp.concatenate([src, jnp.zeros((pad,), jnp.int32)])
    dstp = jnp.concatenate([dst, jnp.full((pad,), NP, jnp.int32)])
    src3d = srcp.reshape(NS * 2, EPW // CH // 2, CH)
    dst3d = dstp.reshape(NS * 2, EPW // CH // 2, CH)
    ones_feat = jnp.ones((CH, D), jnp.float32)
    zrows = jnp.zeros((16, D), jnp.float32)
    flags = jnp.concatenate([jnp.ones((1, 16), jnp.int32),
                             jnp.zeros((1, 16), jnp.int32)])

    w1t = jnp.stack([W1_0[:D], W1_1[:D]])
    w1b = jnp.stack([W1_0[D:], W1_1[D:]])
    b1s = jnp.stack([b1_0, b1_1])
    w2s = jnp.stack([W2_0, W2_1])
    b2s = jnp.stack([b2_0, b2_1])

    def step(carry, xs):
        h, ge_c, dg_c = carry
        w1t_i, w1b_i, b1_i, w2_i, b2_i, fl_i = xs
        out3 = _sc_main(edge_feats, dst3d, h, src3d, zrows, ones_feat, fl_i, NP)
        fresh = fl_i[0] > 0
        ge = jnp.where(fresh, out3[:NP], ge_c)
        gx = out3[NP:2 * NP]
        dg = jnp.where(fresh, out3[2 * NP:], dg_c)
        h_next = _tc_layer(gx, ge, dg, w1t_i, w1b_i, b1_i, w2_i, b2_i, N)
        return (h_next, ge, dg), h_next

    zacc = jnp.zeros((NP, D), jnp.float32)
    (_, _, _), hs = lax.scan(step, (node_feats, zacc, zacc),
                             (w1t, w1b, b1s, w2s, b2s, flags))
    h1, h2 = hs[0], hs[1]
    return _tc_final(node_feats, h1, h2, Wf[:D], Wf[D:2 * D], Wf[2 * D:], bf)


# speedup vs baseline: 2.1996x; 1.1889x over previous
import functools

import jax
import jax.numpy as jnp
from jax import lax
from jax.experimental import pallas as pl
from jax.experimental.pallas import tpu as pltpu
from jax.experimental.pallas import tpu_sc as plsc

SLOPE = 11.0 / 48.0
NS = 16
CH = 80
DEGW = 16


def _sc_main(edge_feats, dst3d, table, src3d, zrows, ones_feat, flag, NP):
    E, D = edge_feats.shape
    HN = NP // 2
    RPH = HN // NS
    NCH = dst3d.shape[0] * dst3d.shape[1] // NS
    SCH = dst3d.shape[1]
    NCHL = (E - (NS - 1) * NCH * CH) // CH
    mesh = plsc.VectorSubcoreMesh(core_axis_name="c", subcore_axis_name="s",
                                  num_cores=1)

    @functools.partial(
        pl.kernel,
        out_type=jax.ShapeDtypeStruct((3 * NP, D), jnp.float32),
        mesh=mesh,
        scratch_types=[
            pltpu.VMEM((16, D), jnp.float32),
            pltpu.VMEM((2 * CH, D), jnp.float32),
            pltpu.VMEM((SCH, CH), jnp.int32),
            pltpu.VMEM((SCH, CH), jnp.int32),
            pltpu.VMEM((16,), jnp.int32),
            pltpu.VMEM_SHARED((HN + 8, D), jnp.float32),
            pltpu.SemaphoreType.DMA,
            pltpu.SemaphoreType.DMA,
            pltpu.SemaphoreType.DMA,
            pltpu.SemaphoreType.DMA,
        ],
    )
    def k(ef_hbm, dst_hbm, tab_hbm, src_hbm, zr_hbm, on_hbm, fl_hbm, out,
          zbuf, pool, dst_v, src_v, fl_v, acc, semA, semB, semC, semD):
        s = lax.axis_index("s")
        r0 = s * RPH
        nch1 = lax.select(s < NS - 1, jnp.int32(NCH), jnp.int32(NCHL))
        pltpu.sync_copy(zr_hbm, zbuf)
        pltpu.sync_copy(fl_hbm, fl_v)
        do_e = fl_v[...][0] > 0

        def zero_acc():
            for q in range(RPH // 16):
                pltpu.sync_copy(zbuf, acc.at[pl.ds(r0 + q * 16, 16)])

        def copyout(sect, lo):
            pltpu.sync_copy(acc.at[pl.ds(r0, RPH)],
                            out.at[pl.ds(sect * NP + lo + r0, RPH)])

        def stage_dst(st, lo):
            pltpu.sync_copy(dst_hbm.at[s * 2 + st], dst_v)

            def rbody(i, carry):
                for j in range(CH // 16):
                    v = dst_v[i, pl.ds(j * 16, 16)]
                    loc = v - lo
                    ok = (loc >= 0) & (loc < HN)
                    dst_v[i, pl.ds(j * 16, 16)] = jnp.where(
                        ok, loc, jnp.full((16,), HN, jnp.int32))
                return carry

            lax.fori_loop(0, SCH, rbody, 0)

        buf0 = pool.at[pl.ds(0, CH)]
        buf1 = pool.at[pl.ds(CH, CH)]

        def pipeline(load_fn, n):
            def body(g, carry):
                c0 = 2 * g
                c1 = 2 * g + 1
                d0 = load_fn(c0, buf0, semA)
                d1 = load_fn(c1, buf1, semB)
                d0.wait()
                e0 = pltpu.async_copy(buf0, acc.at[dst_v.at[c0]], semC,
                                      add=True)
                d1.wait()
                e1 = pltpu.async_copy(buf1, acc.at[dst_v.at[c1]], semD,
                                      add=True)
                e0.wait()
                e1.wait()
                return carry

            lax.fori_loop(0, n // 2, body, 0)

            @pl.when(n % 2 == 1)
            def _():
                load_fn(n - 1, buf0, semA).wait()
                pltpu.async_copy(buf0, acc.at[dst_v.at[n - 1]], semC,
                                 add=True).wait()

        for half in range(2):
            lo = half * HN

            @pl.when(do_e)
            def _():
                zero_acc()
                plsc.subcore_barrier()
                for st in range(2):
                    cb = st * SCH
                    stage_dst(st, lo)
                    nst = jnp.clip(nch1 - cb, 0, SCH)

                    def load_edge(c, b, sem):
                        off = (s * NCH + cb) * CH + c * CH
                        return pltpu.async_copy(ef_hbm.at[pl.ds(off, CH)], b,
                                                sem)

                    pipeline(load_edge, nst)
                plsc.subcore_barrier()
                copyout(0, lo)
                plsc.subcore_barrier()

            zero_acc()
            plsc.subcore_barrier()
            for st in range(2):
                stage_dst(st, lo)
                pltpu.sync_copy(src_hbm.at[s * 2 + st], src_v)

                def load_gather(c, b, sem):
                    return pltpu.async_copy(tab_hbm.at[src_v.at[c]], b, sem)

                pipeline(load_gather, jnp.int32(SCH))
            plsc.subcore_barrier()
            copyout(1, lo)
            plsc.subcore_barrier()

            @pl.when(do_e)
            def _():
                zero_acc()
                pltpu.sync_copy(on_hbm, buf0)
                plsc.subcore_barrier()
                for st in range(2):
                    stage_dst(st, lo)

                    def p3(g, carry):
                        e0 = pltpu.async_copy(buf0, acc.at[dst_v.at[2 * g]],
                                              semC, add=True)
                        e1 = pltpu.async_copy(buf0,
                                              acc.at[dst_v.at[2 * g + 1]],
                                              semD, add=True)
                        e0.wait()
                        e1.wait()
                        return carry

                    lax.fori_loop(0, SCH // 2, p3, 0)

                    @pl.when(SCH % 2 == 1)
                    def _():
                        pltpu.async_copy(buf0, acc.at[dst_v.at[SCH - 1]],
                                         semC, add=True).wait()
                plsc.subcore_barrier()
                copyout(2, lo)
                plsc.subcore_barrier()

    return k(edge_feats, dst3d, table, src3d, zrows, ones_feat, flag)


def _tc_layer(gx, ge, dg, W1t, W1b, b1, W2, b2, N, R=400):
    D = ge.shape[1]
    G = N // R

    def body(gx_r, ge_r, dg_r, w1t_r, w1b_r, b1_r, w2_r, b2_r, out_r):
        deg = dg_r[:, 0:1]
        agg = (jnp.dot(gx_r[...], w1t_r[...], preferred_element_type=jnp.float32)
               + jnp.dot(ge_r[...], w1b_r[...], preferred_element_type=jnp.float32)
               + deg * b1_r[...])
        mean = agg / jnp.maximum(deg, 1.0)
        out = (mean + jnp.dot(mean, w2_r[...], preferred_element_type=jnp.float32)
               + b2_r[...])
        out_r[...] = jnp.where(out >= 0, out, SLOPE * out)

    x_spec = pl.BlockSpec((R, D), lambda i: (i, 0))
    w_spec = pl.BlockSpec((D, D), lambda i: (0, 0))
    b_spec = pl.BlockSpec((1, D), lambda i: (0, 0))
    return pl.pallas_call(
        body,
        grid=(G,),
        in_specs=[x_spec, x_spec, x_spec, w_spec, w_spec, b_spec,
                  w_spec, b_spec],
        out_specs=pl.BlockSpec((R, D), lambda i: (i, 0)),
        out_shape=jax.ShapeDtypeStruct((N, D), jnp.float32),
    )(gx, ge, dg, W1t, W1b, b1.reshape(1, D), W2, b2.reshape(1, D))


def _tc_final(h0, h1, h2, Wf0, Wf1, Wf2, bf, R=400):
    N, D = h0.shape
    G = N // R

    def body(h0_r, h1_r, h2_r, wf0_r, wf1_r, wf2_r, bf_r, out_r):
        out_r[...] = (jnp.dot(h0_r[...], wf0_r[...], preferred_element_type=jnp.float32)
                      + jnp.dot(h1_r[...], wf1_r[...], preferred_element_type=jnp.float32)
                      + jnp.dot(h2_r[...], wf2_r[...], preferred_element_type=jnp.float32)
                      + bf_r[...])

    x_spec = pl.BlockSpec((R, D), lambda i: (i, 0))
    w_spec = pl.BlockSpec((D, D), lambda i: (0, 0))
    b_spec = pl.BlockSpec((1, D), lambda i: (0, 0))
    return pl.pallas_call(
        body,
        grid=(G,),
        in_specs=[x_spec, x_spec, x_spec, w_spec, w_spec, w_spec, b_spec],
        out_specs=pl.BlockSpec((R, D), lambda i: (i, 0)),
        out_shape=jax.ShapeDtypeStruct((N, D), jnp.float32),
    )(h0, h1, h2, Wf0, Wf1, Wf2, bf.reshape(1, D))


def kernel(node_feats, edge_feats, edge_index, W1_0, b1_0, W2_0, b2_0,
           W1_1, b1_1, W2_1, b2_1, Wf, bf):
    N, D = node_feats.shape
    E = edge_feats.shape[0]
    src = edge_index[0]
    dst = edge_index[1]
    NP = ((N + 255) // 256) * 256
    EPW = -(-E // (NS * CH * 2)) * (CH * 2)
    pad = NS * EPW - E
    srcp = jnp.concatenate([src, jnp.zeros((pad,), jnp.int32)])
    dstp = jnp.concatenate([dst, jnp.full((pad,), NP, jnp.int32)])
    src3d = srcp.reshape(NS * 2, EPW // CH // 2, CH)
    dst3d = dstp.reshape(NS * 2, EPW // CH // 2, CH)
    ones_feat = jnp.ones((CH, D), jnp.float32)
    zrows = jnp.zeros((16, D), jnp.float32)
    flags = jnp.concatenate([jnp.ones((1, 16), jnp.int32),
                             jnp.zeros((1, 16), jnp.int32)])

    w1t = jnp.stack([W1_0[:D], W1_1[:D]])
    w1b = jnp.stack([W1_0[D:], W1_1[D:]])
    b1s = jnp.stack([b1_0, b1_1])
    w2s = jnp.stack([W2_0, W2_1])
    b2s = jnp.stack([b2_0, b2_1])

    def step(carry, xs):
        h, ge_c, dg_c = carry
        w1t_i, w1b_i, b1_i, w2_i, b2_i, fl_i = xs
        out3 = _sc_main(edge_feats, dst3d, h, src3d, zrows, ones_feat, fl_i, NP)
        fresh = fl_i[0] > 0
        ge = jnp.where(fresh, out3[:NP], ge_c)
        gx = out3[NP:2 * NP]
        dg = jnp.where(fresh, out3[2 * NP:], dg_c)
        h_next = _tc_layer(gx, ge, dg, w1t_i, w1b_i, b1_i, w2_i, b2_i, N)
        return (h_next, ge, dg), h_next

    zacc = jnp.zeros((NP, D), jnp.float32)
    (_, _, _), hs = lax.scan(step, (node_feats, zacc, zacc),
                             (w1t, w1b, b1s, w2s, b2s, flags))
    h1, h2 = hs[0], hs[1]
    return _tc_final(node_feats, h1, h2, Wf[:D], Wf[D:2 * D], Wf[2 * D:], bf)


# speedup vs baseline: 2.3094x; 1.0499x over previous
import functools

import jax
import jax.numpy as jnp
from jax import lax
from jax.experimental import pallas as pl
from jax.experimental.pallas import tpu as pltpu
from jax.experimental.pallas import tpu_sc as plsc

SLOPE = 11.0 / 48.0
NS = 16
CH = 80
DEGW = 16


def _sc_main(edge_feats, dst3d, table, src3d, zrows, ones_feat, flag, NP):
    E, D = edge_feats.shape
    HN = NP // 2
    RPH = HN // NS
    NCH = dst3d.shape[0] * dst3d.shape[1] // NS
    SCH = dst3d.shape[1]
    NCHL = (E - (NS - 1) * NCH * CH) // CH
    mesh = plsc.VectorSubcoreMesh(core_axis_name="c", subcore_axis_name="s",
                                  num_cores=1)

    @functools.partial(
        pl.kernel,
        out_type=jax.ShapeDtypeStruct((3 * NP, D), jnp.float32),
        mesh=mesh,
        scratch_types=[
            pltpu.VMEM((16, D), jnp.float32),
            pltpu.VMEM((4 * CH, D), jnp.float32),
            pltpu.VMEM((SCH, CH), jnp.int32),
            pltpu.VMEM((SCH, CH), jnp.int32),
            pltpu.VMEM((16,), jnp.int32),
            pltpu.VMEM_SHARED((HN + 8, D), jnp.float32),
            [pltpu.SemaphoreType.DMA] * 8,
        ],
    )
    def k(ef_hbm, dst_hbm, tab_hbm, src_hbm, zr_hbm, on_hbm, fl_hbm, out,
          zbuf, pool, dst_v, src_v, fl_v, acc, sems):
        s = lax.axis_index("s")
        r0 = s * RPH
        nch1 = lax.select(s < NS - 1, jnp.int32(NCH), jnp.int32(NCHL))
        pltpu.sync_copy(zr_hbm, zbuf)
        pltpu.sync_copy(fl_hbm, fl_v)
        do_e = fl_v[...][0] > 0

        def zero_acc():
            for q in range(RPH // 16):
                pltpu.sync_copy(zbuf, acc.at[pl.ds(r0 + q * 16, 16)])

        def copyout(sect, lo):
            pltpu.sync_copy(acc.at[pl.ds(r0, RPH)],
                            out.at[pl.ds(sect * NP + lo + r0, RPH)])

        def stage_dst(st, lo):
            pltpu.sync_copy(dst_hbm.at[s * 2 + st], dst_v)

            def rbody(i, carry):
                for j in range(CH // 16):
                    v = dst_v[i, pl.ds(j * 16, 16)]
                    loc = v - lo
                    ok = (loc >= 0) & (loc < HN)
                    dst_v[i, pl.ds(j * 16, 16)] = jnp.where(
                        ok, loc, jnp.full((16,), HN, jnp.int32))
                return carry

            lax.fori_loop(0, SCH, rbody, 0)

        bufs = [pool.at[pl.ds(j * CH, CH)] for j in range(4)]
        buf0 = bufs[0]
        semC = sems[4]
        semD = sems[5]

        def pipeline(load_fn, n):
            def body(g, carry):
                ds_ = []
                for j in range(4):
                    ds_.append(load_fn(4 * g + j, bufs[j], sems[j]))
                es = []
                for j in range(4):
                    ds_[j].wait()
                    es.append(pltpu.async_copy(
                        bufs[j], acc.at[dst_v.at[4 * g + j]], sems[4 + j],
                        add=True))
                for e in es:
                    e.wait()
                return carry

            lax.fori_loop(0, n // 4, body, 0)

            def tail(i, carry):
                load_fn(i, bufs[0], sems[0]).wait()
                pltpu.async_copy(bufs[0], acc.at[dst_v.at[i]], sems[4],
                                 add=True).wait()
                return carry

            lax.fori_loop((n // 4) * 4, n, tail, 0)

        for half in range(2):
            lo = half * HN

            @pl.when(do_e)
            def _():
                zero_acc()
                plsc.subcore_barrier()
                for st in range(2):
                    cb = st * SCH
                    stage_dst(st, lo)
                    nst = jnp.clip(nch1 - cb, 0, SCH)

                    def load_edge(c, b, sem):
                        off = (s * NCH + cb) * CH + c * CH
                        return pltpu.async_copy(ef_hbm.at[pl.ds(off, CH)], b,
                                                sem)

                    pipeline(load_edge, nst)
                plsc.subcore_barrier()
                copyout(0, lo)
                plsc.subcore_barrier()

            zero_acc()
            plsc.subcore_barrier()
            for st in range(2):
                stage_dst(st, lo)
                pltpu.sync_copy(src_hbm.at[s * 2 + st], src_v)

                def load_gather(c, b, sem):
                    return pltpu.async_copy(tab_hbm.at[src_v.at[c]], b, sem)

                pipeline(load_gather, jnp.int32(SCH))
            plsc.subcore_barrier()
            copyout(1, lo)
            plsc.subcore_barrier()

            @pl.when(do_e)
            def _():
                zero_acc()
                pltpu.sync_copy(on_hbm, buf0)
                plsc.subcore_barrier()
                for st in range(2):
                    stage_dst(st, lo)

                    def p3(g, carry):
                        e0 = pltpu.async_copy(buf0, acc.at[dst_v.at[2 * g]],
                                              semC, add=True)
                        e1 = pltpu.async_copy(buf0,
                                              acc.at[dst_v.at[2 * g + 1]],
                                              semD, add=True)
                        e0.wait()
                        e1.wait()
                        return carry

                    lax.fori_loop(0, SCH // 2, p3, 0)

                    @pl.when(SCH % 2 == 1)
                    def _():
                        pltpu.async_copy(buf0, acc.at[dst_v.at[SCH - 1]],
                                         semC, add=True).wait()
                plsc.subcore_barrier()
                copyout(2, lo)
                plsc.subcore_barrier()

    return k(edge_feats, dst3d, table, src3d, zrows, ones_feat, flag)


def _tc_layer(gx, ge, dg, W1t, W1b, b1, W2, b2, N, R=400):
    D = ge.shape[1]
    G = N // R

    def body(gx_r, ge_r, dg_r, w1t_r, w1b_r, b1_r, w2_r, b2_r, out_r):
        deg = dg_r[:, 0:1]
        agg = (jnp.dot(gx_r[...], w1t_r[...], preferred_element_type=jnp.float32)
               + jnp.dot(ge_r[...], w1b_r[...], preferred_element_type=jnp.float32)
               + deg * b1_r[...])
        mean = agg / jnp.maximum(deg, 1.0)
        out = (mean + jnp.dot(mean, w2_r[...], preferred_element_type=jnp.float32)
               + b2_r[...])
        out_r[...] = jnp.where(out >= 0, out, SLOPE * out)

    x_spec = pl.BlockSpec((R, D), lambda i: (i, 0))
    w_spec = pl.BlockSpec((D, D), lambda i: (0, 0))
    b_spec = pl.BlockSpec((1, D), lambda i: (0, 0))
    return pl.pallas_call(
        body,
        grid=(G,),
        in_specs=[x_spec, x_spec, x_spec, w_spec, w_spec, b_spec,
                  w_spec, b_spec],
        out_specs=pl.BlockSpec((R, D), lambda i: (i, 0)),
        out_shape=jax.ShapeDtypeStruct((N, D), jnp.float32),
    )(gx, ge, dg, W1t, W1b, b1.reshape(1, D), W2, b2.reshape(1, D))


def _tc_final(h0, h1, h2, Wf0, Wf1, Wf2, bf, R=400):
    N, D = h0.shape
    G = N // R

    def body(h0_r, h1_r, h2_r, wf0_r, wf1_r, wf2_r, bf_r, out_r):
        out_r[...] = (jnp.dot(h0_r[...], wf0_r[...], preferred_element_type=jnp.float32)
                      + jnp.dot(h1_r[...], wf1_r[...], preferred_element_type=jnp.float32)
                      + jnp.dot(h2_r[...], wf2_r[...], preferred_element_type=jnp.float32)
                      + bf_r[...])

    x_spec = pl.BlockSpec((R, D), lambda i: (i, 0))
    w_spec = pl.BlockSpec((D, D), lambda i: (0, 0))
    b_spec = pl.BlockSpec((1, D), lambda i: (0, 0))
    return pl.pallas_call(
        body,
        grid=(G,),
        in_specs=[x_spec, x_spec, x_spec, w_spec, w_spec, w_spec, b_spec],
        out_specs=pl.BlockSpec((R, D), lambda i: (i, 0)),
        out_shape=jax.ShapeDtypeStruct((N, D), jnp.float32),
    )(h0, h1, h2, Wf0, Wf1, Wf2, bf.reshape(1, D))


def kernel(node_feats, edge_feats, edge_index, W1_0, b1_0, W2_0, b2_0,
           W1_1, b1_1, W2_1, b2_1, Wf, bf):
    N, D = node_feats.shape
    E = edge_feats.shape[0]
    src = edge_index[0]
    dst = edge_index[1]
    NP = ((N + 255) // 256) * 256
    EPW = -(-E // (NS * CH * 2)) * (CH * 2)
    pad = NS * EPW - E
    srcp = jnp.concatenate([src, jnp.zeros((pad,), jnp.int32)])
    dstp = jnp.concatenate([dst, jnp.full((pad,), NP, jnp.int32)])
    src3d = srcp.reshape(NS * 2, EPW // CH // 2, CH)
    dst3d = dstp.reshape(NS * 2, EPW // CH // 2, CH)
    ones_feat = jnp.ones((CH, D), jnp.float32)
    zrows = jnp.zeros((16, D), jnp.float32)
    flags = jnp.concatenate([jnp.ones((1, 16), jnp.int32),
                             jnp.zeros((1, 16), jnp.int32)])

    w1t = jnp.stack([W1_0[:D], W1_1[:D]])
    w1b = jnp.stack([W1_0[D:], W1_1[D:]])
    b1s = jnp.stack([b1_0, b1_1])
    w2s = jnp.stack([W2_0, W2_1])
    b2s = jnp.stack([b2_0, b2_1])

    def step(carry, xs):
        h, ge_c, dg_c = carry
        w1t_i, w1b_i, b1_i, w2_i, b2_i, fl_i = xs
        out3 = _sc_main(edge_feats, dst3d, h, src3d, zrows, ones_feat, fl_i, NP)
        fresh = fl_i[0] > 0
        ge = jnp.where(fresh, out3[:NP], ge_c)
        gx = out3[NP:2 * NP]
        dg = jnp.where(fresh, out3[2 * NP:], dg_c)
        h_next = _tc_layer(gx, ge, dg, w1t_i, w1b_i, b1_i, w2_i, b2_i, N)
        return (h_next, ge, dg), h_next

    zacc = jnp.zeros((NP, D), jnp.float32)
    (_, _, _), hs = lax.scan(step, (node_feats, zacc, zacc),
                             (w1t, w1b, b1s, w2s, b2s, flags))
    h1, h2 = hs[0], hs[1]
    return _tc_final(node_feats, h1, h2, Wf[:D], Wf[D:2 * D], Wf[2 * D:], bf)
